# baseline (speedup 1.0000x reference)
import functools

import jax
import jax.numpy as jnp
from jax import lax
from jax.experimental import pallas as pl
from jax.experimental.pallas import tpu as pltpu
from jax.experimental.pallas import tpu_sc as plsc

EMBED = 64
NC = 2
NS = 16
NW = NC * NS

CHUNK = 800


@functools.lru_cache(maxsize=None)
def _build(B):
    b_per_w = B // NW
    nchunks = b_per_w // CHUNK
    assert b_per_w % CHUNK == 0

    mesh = plsc.VectorSubcoreMesh(core_axis_name="c", subcore_axis_name="s")

    @functools.partial(
        pl.kernel,
        mesh=mesh,
        out_type=jax.ShapeDtypeStruct((B, EMBED), jnp.float32),
        compiler_params=pltpu.CompilerParams(use_tc_tiling_on_sc=False),
        scratch_types=[
            pltpu.VMEM((b_per_w,), jnp.int32),
            pltpu.VMEM((CHUNK, EMBED), jnp.float32),
            pltpu.SemaphoreType.DMA,
        ],
    )
    def k(table_hbm, idx_hbm, out_hbm, idx_v, rows, sg):
        wid = lax.axis_index("s") * NC + lax.axis_index("c")
        base = wid * b_per_w
        pltpu.sync_copy(idx_hbm.at[pl.ds(base, b_per_w)], idx_v)

        def body(g, carry):
            off = pl.multiple_of(g * CHUNK, 8)
            pltpu.async_copy(
                table_hbm.at[idx_v.at[pl.ds(off, CHUNK)]], rows, sg
            ).wait()
            pltpu.sync_copy(rows, out_hbm.at[pl.ds(base + off, CHUNK)])
            return carry

        lax.fori_loop(0, nchunks, body, 0)

    return k


@jax.jit
def kernel(x, table):
    b, t = x.shape
    flat = x.reshape(b * t)
    out = _build(b * t)(table, flat)
    return out.reshape(b, t, EMBED)
